# baseline (speedup 1.0000x reference)
import functools

import jax
import jax.numpy as jnp
from jax import lax
from jax.experimental import pallas as pl
from jax.experimental.pallas import tpu as pltpu
from jax.experimental.pallas import tpu_sc as plsc

N_NODES = 10000
N_EDGES = 320000
DIM = 128

NC = 2
NS = 16
NW = NC * NS

NP = 10240
EB = 128
EP = 327680
NB = EP // (NW * EB)
CH = 40
RPT = NP // NS

_MESH = plsc.VectorSubcoreMesh(
    core_axis_name="c", subcore_axis_name="s", num_cores=NC, num_subcores=NS
)
_SC_PARAMS = pltpu.CompilerParams(needs_layout_passes=False)


@functools.partial(
    pl.kernel,
    out_type=jax.ShapeDtypeStruct((NW * NP,), jnp.float32),
    mesh=_MESH,
    scratch_types=[
        pltpu.VMEM((NB, EB), jnp.int32),
        pltpu.VMEM((NP,), jnp.float32),
    ],
    compiler_params=_SC_PARAMS,
)
def _deg_kernel(dst_hbm, out_hbm, dst_v, deg_v):
    c = lax.axis_index("c")
    s = lax.axis_index("s")
    wid = c * NS + s
    pltpu.sync_copy(dst_hbm.at[wid], dst_v)

    def zero_body(r, _):
        deg_v[pl.ds(r * 16, 16)] = jnp.zeros((16,), jnp.float32)
        return 0

    lax.fori_loop(0, NP // 16, zero_body, 0, unroll=False)

    ones = jnp.ones((16,), jnp.float32)
    chunks = NB * EB // 16

    def body(i, _):
        r = i // (EB // 16)
        k = i % (EB // 16)
        idx = dst_v[r, pl.ds(k * 16, 16)]
        plsc.addupdate_scatter(deg_v, [idx], ones)
        return 0

    lax.fori_loop(0, chunks, body, 0, unroll=False)
    pltpu.sync_copy(deg_v, out_hbm.at[pl.ds(wid * NP, NP)])


@functools.partial(
    pl.kernel,
    out_type=jax.ShapeDtypeStruct((NC, NP, DIM), jnp.float32),
    mesh=_MESH,
    scratch_types=[
        pltpu.VMEM((CH, EB), jnp.int32),
        pltpu.VMEM((CH, EB), jnp.int32),
        pltpu.VMEM((EB, DIM), jnp.float32),
        pltpu.VMEM((EB, DIM), jnp.float32),
        pltpu.VMEM_SHARED((NP, DIM), jnp.float32),
        pltpu.SemaphoreType.DMA,
        pltpu.SemaphoreType.DMA,
    ],
    compiler_params=_SC_PARAMS,
)
def _agg_kernel(hp_hbm, src_hbm, dst_hbm, out_hbm,
                src_v, dst_v, rows0, rows1, acc, sg0, sg1):
    c = lax.axis_index("c")
    s = lax.axis_index("s")
    wid = c * NS + s

    def zero_body(r, _):
        for k in range(DIM // 16):
            rows0[r, pl.ds(k * 16, 16)] = jnp.zeros((16,), jnp.float32)
        return 0

    lax.fori_loop(0, EB, zero_body, 0, unroll=False)
    for q in range(RPT // EB):
        pltpu.sync_copy(rows0, acc.at[pl.ds(s * RPT + q * EB, EB)])
    plsc.subcore_barrier()

    for q in range(NB // CH):
        pltpu.sync_copy(src_hbm.at[wid, pl.ds(q * CH, CH)], src_v)
        pltpu.sync_copy(dst_hbm.at[wid, pl.ds(q * CH, CH)], dst_v)
        pltpu.async_copy(hp_hbm.at[src_v.at[0]], rows0, sg0)

        def body(j, _):
            even = j % 2 == 0

            @pl.when(jnp.logical_and(even, j + 1 < CH))
            def _():
                pltpu.async_copy(hp_hbm.at[src_v.at[j + 1]], rows1, sg1)

            @pl.when(jnp.logical_and(jnp.logical_not(even), j + 1 < CH))
            def _():
                pltpu.async_copy(hp_hbm.at[src_v.at[j + 1]], rows0, sg0)

            @pl.when(even)
            def _():
                pltpu.make_async_copy(
                    hp_hbm.at[src_v.at[j]], rows0, sg0).wait()
                pltpu.sync_copy(rows0, acc.at[dst_v.at[j]], add=True)

            @pl.when(jnp.logical_not(even))
            def _():
                pltpu.make_async_copy(
                    hp_hbm.at[src_v.at[j]], rows1, sg1).wait()
                pltpu.sync_copy(rows1, acc.at[dst_v.at[j]], add=True)

            return 0

        lax.fori_loop(0, CH, body, 0, unroll=False)

    plsc.subcore_barrier()
    pltpu.sync_copy(acc.at[pl.ds(s * RPT, RPT)],
                    out_hbm.at[c, pl.ds(s * RPT, RPT)])


_RB = 640


def _mm_body(x_ref, w_ref, h_ref):
    h_ref[...] = jnp.dot(x_ref[...], w_ref[...],
                         preferred_element_type=jnp.float32)


def _mm_tc(x, w):
    grid = (NP // _RB,)
    return pl.pallas_call(
        _mm_body,
        grid=grid,
        in_specs=[
            pl.BlockSpec((_RB, DIM), lambda i: (i, 0)),
            pl.BlockSpec((DIM, DIM), lambda i: (0, 0)),
        ],
        out_specs=pl.BlockSpec((_RB, DIM), lambda i: (i, 0)),
        out_shape=jax.ShapeDtypeStruct((NP, DIM), jnp.float32),
    )(x, w)


def _scale_body(h_ref, degp_ref, hp_ref, dis_ref):
    deg = jnp.sum(degp_ref[...], axis=0) + 1.0
    dis = lax.rsqrt(deg)[:, None]
    hp_ref[...] = h_ref[...] * dis
    dis_ref[...] = dis


def _scale_tc(h, degp):
    grid = (NP // _RB,)
    return pl.pallas_call(
        _scale_body,
        grid=grid,
        in_specs=[
            pl.BlockSpec((_RB, DIM), lambda i: (i, 0)),
            pl.BlockSpec((NW, _RB), lambda i: (0, i)),
        ],
        out_specs=[
            pl.BlockSpec((_RB, DIM), lambda i: (i, 0)),
            pl.BlockSpec((_RB, 1), lambda i: (i, 0)),
        ],
        out_shape=[
            jax.ShapeDtypeStruct((NP, DIM), jnp.float32),
            jax.ShapeDtypeStruct((NP, 1), jnp.float32),
        ],
    )(h, degp)


def _mid_body(p_ref, hp_ref, dis_ref, b_ref, w_ref, out_ref):
    z = (p_ref[0] + p_ref[1] + hp_ref[...]) * dis_ref[...] + b_ref[...]
    x2 = jnp.maximum(z, 0.0)
    h2 = jnp.dot(x2, w_ref[...], preferred_element_type=jnp.float32)
    out_ref[...] = h2 * dis_ref[...]


def _mid_tc(p, hp, dis, b, w):
    grid = (NP // _RB,)
    return pl.pallas_call(
        _mid_body,
        grid=grid,
        in_specs=[
            pl.BlockSpec((NC, _RB, DIM), lambda i: (0, i, 0)),
            pl.BlockSpec((_RB, DIM), lambda i: (i, 0)),
            pl.BlockSpec((_RB, 1), lambda i: (i, 0)),
            pl.BlockSpec((1, DIM), lambda i: (0, 0)),
            pl.BlockSpec((DIM, DIM), lambda i: (0, 0)),
        ],
        out_specs=pl.BlockSpec((_RB, DIM), lambda i: (i, 0)),
        out_shape=jax.ShapeDtypeStruct((NP, DIM), jnp.float32),
    )(p, hp, dis, b, w)


def _fin_body(p_ref, hp_ref, dis_ref, b_ref, out_ref):
    out_ref[...] = (p_ref[0] + p_ref[1] + hp_ref[...]) * dis_ref[...] \
        + b_ref[...]


_FB = 1000


def _fin_tc(p, hp, dis, b):
    grid = (N_NODES // _FB,)
    return pl.pallas_call(
        _fin_body,
        grid=grid,
        in_specs=[
            pl.BlockSpec((NC, _FB, DIM), lambda i: (0, i, 0)),
            pl.BlockSpec((_FB, DIM), lambda i: (i, 0)),
            pl.BlockSpec((_FB, 1), lambda i: (i, 0)),
            pl.BlockSpec((1, DIM), lambda i: (0, 0)),
        ],
        out_specs=pl.BlockSpec((_FB, DIM), lambda i: (i, 0)),
        out_shape=jax.ShapeDtypeStruct((N_NODES, DIM), jnp.float32),
    )(p, hp, dis, b)


@jax.jit
def kernel(x, edge_index, W1, b1, W2, b2):
    pad = N_NODES + jnp.arange(EP - N_EDGES, dtype=jnp.int32) % (NP - N_NODES)
    src = jnp.concatenate([edge_index[0].astype(jnp.int32), pad])
    dst = jnp.concatenate([edge_index[1].astype(jnp.int32), pad])
    src3d = src.reshape(NW, NB, EB)
    dst3d = dst.reshape(NW, NB, EB)
    xp = jnp.pad(x, ((0, NP - N_NODES), (0, 0)))

    degp = _deg_kernel(dst3d).reshape(NW, NP)
    h1 = _mm_tc(xp, W1)
    hp1, dis = _scale_tc(h1, degp)
    p1 = _agg_kernel(hp1, src3d, dst3d)
    hp2 = _mid_tc(p1, hp1, dis, b1.reshape(1, DIM), W2)
    p2 = _agg_kernel(hp2, src3d, dst3d)
    return _fin_tc(p2, hp2, dis, b2.reshape(1, DIM))


# speedup vs baseline: 1.0262x; 1.0262x over previous
import functools

import jax
import jax.numpy as jnp
from jax import lax
from jax.experimental import pallas as pl
from jax.experimental.pallas import tpu as pltpu
from jax.experimental.pallas import tpu_sc as plsc

N_NODES = 10000
N_EDGES = 320000
DIM = 128

NC = 2
NS = 16
NW = NC * NS

NP = 10240
EB = 128
EP = 327680
NB = EP // (NW * EB)
CH = 40
RPT = NP // NS

_MESH = plsc.VectorSubcoreMesh(
    core_axis_name="c", subcore_axis_name="s", num_cores=NC, num_subcores=NS
)
_SC_PARAMS = pltpu.CompilerParams(needs_layout_passes=False)


@functools.partial(
    pl.kernel,
    out_type=jax.ShapeDtypeStruct((NW * NP,), jnp.float32),
    mesh=_MESH,
    scratch_types=[
        pltpu.VMEM((NB, EB), jnp.int32),
        pltpu.VMEM((NP,), jnp.float32),
    ],
    compiler_params=_SC_PARAMS,
)
def _deg_kernel(dst_hbm, out_hbm, dst_v, deg_v):
    c = lax.axis_index("c")
    s = lax.axis_index("s")
    wid = c * NS + s
    pltpu.sync_copy(dst_hbm.at[wid], dst_v)

    def zero_body(r, _):
        deg_v[pl.ds(r * 16, 16)] = jnp.zeros((16,), jnp.float32)
        return 0

    lax.fori_loop(0, NP // 16, zero_body, 0, unroll=False)

    ones = jnp.ones((16,), jnp.float32)

    def body(r, _):
        for k in range(EB // 16):
            idx = dst_v[r, pl.ds(k * 16, 16)]
            plsc.addupdate_scatter(deg_v, [idx], ones)
        return 0

    lax.fori_loop(0, NB, body, 0, unroll=False)
    pltpu.sync_copy(deg_v, out_hbm.at[pl.ds(wid * NP, NP)])


@functools.partial(
    pl.kernel,
    out_type=jax.ShapeDtypeStruct((NC, NP, DIM), jnp.float32),
    mesh=_MESH,
    scratch_types=[
        pltpu.VMEM((CH, EB), jnp.int32),
        pltpu.VMEM((CH, EB), jnp.int32),
        pltpu.VMEM((EB, DIM), jnp.float32),
        pltpu.VMEM((EB, DIM), jnp.float32),
        pltpu.VMEM_SHARED((NP, DIM), jnp.float32),
        pltpu.SemaphoreType.DMA,
        pltpu.SemaphoreType.DMA,
    ],
    compiler_params=_SC_PARAMS,
)
def _agg_kernel(hp_hbm, src_hbm, dst_hbm, out_hbm,
                src_v, dst_v, rows0, rows1, acc, sg0, sg1):
    c = lax.axis_index("c")
    s = lax.axis_index("s")
    wid = c * NS + s

    pltpu.async_copy(src_hbm.at[wid, pl.ds(0, CH)], src_v, sg0)
    pltpu.async_copy(dst_hbm.at[wid, pl.ds(0, CH)], dst_v, sg1)

    def zero_body(r, _):
        for k in range(DIM // 16):
            rows0[r, pl.ds(k * 16, 16)] = jnp.zeros((16,), jnp.float32)
        return 0

    lax.fori_loop(0, EB, zero_body, 0, unroll=False)
    for q in range(RPT // EB):
        pltpu.sync_copy(rows0, acc.at[pl.ds(s * RPT + q * EB, EB)])
    pltpu.make_async_copy(src_hbm.at[wid, pl.ds(0, CH)], src_v, sg0).wait()
    pltpu.make_async_copy(dst_hbm.at[wid, pl.ds(0, CH)], dst_v, sg1).wait()
    plsc.subcore_barrier()

    for q in range(NB // CH):
        if q > 0:
            pltpu.sync_copy(src_hbm.at[wid, pl.ds(q * CH, CH)], src_v)
            pltpu.sync_copy(dst_hbm.at[wid, pl.ds(q * CH, CH)], dst_v)
        pltpu.async_copy(hp_hbm.at[src_v.at[0]], rows0, sg0)

        def body(j, _):
            even = j % 2 == 0

            @pl.when(jnp.logical_and(even, j + 1 < CH))
            def _():
                pltpu.async_copy(hp_hbm.at[src_v.at[j + 1]], rows1, sg1)

            @pl.when(jnp.logical_and(jnp.logical_not(even), j + 1 < CH))
            def _():
                pltpu.async_copy(hp_hbm.at[src_v.at[j + 1]], rows0, sg0)

            @pl.when(even)
            def _():
                pltpu.make_async_copy(
                    hp_hbm.at[src_v.at[j]], rows0, sg0).wait()
                pltpu.sync_copy(rows0, acc.at[dst_v.at[j]], add=True)

            @pl.when(jnp.logical_not(even))
            def _():
                pltpu.make_async_copy(
                    hp_hbm.at[src_v.at[j]], rows1, sg1).wait()
                pltpu.sync_copy(rows1, acc.at[dst_v.at[j]], add=True)

            return 0

        lax.fori_loop(0, CH, body, 0, unroll=False)

    plsc.subcore_barrier()
    pltpu.sync_copy(acc.at[pl.ds(s * RPT, RPT)],
                    out_hbm.at[c, pl.ds(s * RPT, RPT)])


_RB = 640


def _prep_body(x_ref, w_ref, degp_ref, hp_ref, dis_ref):
    deg = jnp.sum(degp_ref[...], axis=0) + 1.0
    dis = lax.rsqrt(deg)[:, None]
    h = jnp.dot(x_ref[...], w_ref[...], preferred_element_type=jnp.float32)
    hp_ref[...] = h * dis
    dis_ref[...] = dis


def _prep_tc(x, w, degp):
    grid = (NP // _RB,)
    return pl.pallas_call(
        _prep_body,
        grid=grid,
        in_specs=[
            pl.BlockSpec((_RB, DIM), lambda i: (i, 0)),
            pl.BlockSpec((DIM, DIM), lambda i: (0, 0)),
            pl.BlockSpec((NW, _RB), lambda i: (0, i)),
        ],
        out_specs=[
            pl.BlockSpec((_RB, DIM), lambda i: (i, 0)),
            pl.BlockSpec((_RB, 1), lambda i: (i, 0)),
        ],
        out_shape=[
            jax.ShapeDtypeStruct((NP, DIM), jnp.float32),
            jax.ShapeDtypeStruct((NP, 1), jnp.float32),
        ],
    )(x, w, degp)


def _mid_body(p_ref, hp_ref, dis_ref, b_ref, w_ref, out_ref):
    z = (p_ref[0] + p_ref[1] + hp_ref[...]) * dis_ref[...] + b_ref[...]
    x2 = jnp.maximum(z, 0.0)
    h2 = jnp.dot(x2, w_ref[...], preferred_element_type=jnp.float32)
    out_ref[...] = h2 * dis_ref[...]


def _mid_tc(p, hp, dis, b, w):
    grid = (NP // _RB,)
    return pl.pallas_call(
        _mid_body,
        grid=grid,
        in_specs=[
            pl.BlockSpec((NC, _RB, DIM), lambda i: (0, i, 0)),
            pl.BlockSpec((_RB, DIM), lambda i: (i, 0)),
            pl.BlockSpec((_RB, 1), lambda i: (i, 0)),
            pl.BlockSpec((1, DIM), lambda i: (0, 0)),
            pl.BlockSpec((DIM, DIM), lambda i: (0, 0)),
        ],
        out_specs=pl.BlockSpec((_RB, DIM), lambda i: (i, 0)),
        out_shape=jax.ShapeDtypeStruct((NP, DIM), jnp.float32),
    )(p, hp, dis, b, w)


def _fin_body(p_ref, hp_ref, dis_ref, b_ref, out_ref):
    out_ref[...] = (p_ref[0] + p_ref[1] + hp_ref[...]) * dis_ref[...] \
        + b_ref[...]


_FB = 1000


def _fin_tc(p, hp, dis, b):
    grid = (N_NODES // _FB,)
    return pl.pallas_call(
        _fin_body,
        grid=grid,
        in_specs=[
            pl.BlockSpec((NC, _FB, DIM), lambda i: (0, i, 0)),
            pl.BlockSpec((_FB, DIM), lambda i: (i, 0)),
            pl.BlockSpec((_FB, 1), lambda i: (i, 0)),
            pl.BlockSpec((1, DIM), lambda i: (0, 0)),
        ],
        out_specs=pl.BlockSpec((_FB, DIM), lambda i: (i, 0)),
        out_shape=jax.ShapeDtypeStruct((N_NODES, DIM), jnp.float32),
    )(p, hp, dis, b)


@jax.jit
def kernel(x, edge_index, W1, b1, W2, b2):
    pad = N_NODES + jnp.arange(EP - N_EDGES, dtype=jnp.int32) % (NP - N_NODES)
    src = jnp.concatenate([edge_index[0].astype(jnp.int32), pad])
    dst = jnp.concatenate([edge_index[1].astype(jnp.int32), pad])
    src3d = src.reshape(NW, NB, EB)
    dst3d = dst.reshape(NW, NB, EB)
    xp = jnp.pad(x, ((0, NP - N_NODES), (0, 0)))

    degp = _deg_kernel(dst3d).reshape(NW, NP)
    hp1, dis = _prep_tc(xp, W1, degp)
    p1 = _agg_kernel(hp1, src3d, dst3d)
    hp2 = _mid_tc(p1, hp1, dis, b1.reshape(1, DIM), W2)
    p2 = _agg_kernel(hp2, src3d, dst3d)
    return _fin_tc(p2, hp2, dis, b2.reshape(1, DIM))


# speedup vs baseline: 1.0295x; 1.0032x over previous
import functools

import jax
import jax.numpy as jnp
from jax import lax
from jax.experimental import pallas as pl
from jax.experimental.pallas import tpu as pltpu
from jax.experimental.pallas import tpu_sc as plsc

N_NODES = 10000
N_EDGES = 320000
DIM = 128

NC = 2
NS = 16
NW = NC * NS

NP = 10240
EB = 128
EP = 327680
NB = EP // (NW * EB)
CH = 40
RPT = NP // NS

_MESH = plsc.VectorSubcoreMesh(
    core_axis_name="c", subcore_axis_name="s", num_cores=NC, num_subcores=NS
)
_SC_PARAMS = pltpu.CompilerParams(needs_layout_passes=False)


@functools.partial(
    pl.kernel,
    out_type=jax.ShapeDtypeStruct((NW * NP,), jnp.float32),
    mesh=_MESH,
    scratch_types=[
        pltpu.VMEM((NB, EB), jnp.int32),
        pltpu.VMEM((NP,), jnp.float32),
    ],
    compiler_params=_SC_PARAMS,
)
def _deg_kernel(dst_hbm, out_hbm, dst_v, deg_v):
    c = lax.axis_index("c")
    s = lax.axis_index("s")
    wid = c * NS + s
    pltpu.sync_copy(dst_hbm.at[wid], dst_v)

    def zero_body(r, _):
        deg_v[pl.ds(r * 16, 16)] = jnp.zeros((16,), jnp.float32)
        return 0

    lax.fori_loop(0, NP // 16, zero_body, 0, unroll=False)

    ones = jnp.ones((16,), jnp.float32)

    def body(r, _):
        for k in range(EB // 16):
            idx = dst_v[r, pl.ds(k * 16, 16)]
            plsc.addupdate_scatter(deg_v, [idx], ones)
        return 0

    lax.fori_loop(0, NB, body, 0, unroll=False)
    pltpu.sync_copy(deg_v, out_hbm.at[pl.ds(wid * NP, NP)])


@functools.partial(
    pl.kernel,
    out_type=jax.ShapeDtypeStruct((NC, NP, DIM), jnp.float32),
    mesh=_MESH,
    scratch_types=[
        pltpu.VMEM((CH, EB), jnp.int32),
        pltpu.VMEM((CH, EB), jnp.int32),
        pltpu.VMEM((EB, DIM), jnp.float32),
        pltpu.VMEM((EB, DIM), jnp.float32),
        pltpu.VMEM_SHARED((NP, DIM), jnp.float32),
        pltpu.SemaphoreType.DMA,
        pltpu.SemaphoreType.DMA,
        pltpu.SemaphoreType.DMA,
    ],
    compiler_params=_SC_PARAMS,
)
def _agg_kernel(hp_hbm, src_hbm, dst_hbm, out_hbm,
                src_v, dst_v, rows0, rows1, acc, sg0, sg1, sz):
    c = lax.axis_index("c")
    s = lax.axis_index("s")
    wid = c * NS + s

    pltpu.async_copy(src_hbm.at[wid, pl.ds(0, CH)], src_v, sg0)
    pltpu.async_copy(dst_hbm.at[wid, pl.ds(0, CH)], dst_v, sg1)

    def zero_body(r, _):
        for k in range(DIM // 16):
            rows0[r, pl.ds(k * 16, 16)] = jnp.zeros((16,), jnp.float32)
        return 0

    lax.fori_loop(0, EB, zero_body, 0, unroll=False)
    for q in range(RPT // EB):
        pltpu.async_copy(rows0, acc.at[pl.ds(s * RPT + q * EB, EB)], sz)
    for q in range(RPT // EB):
        pltpu.make_async_copy(rows0, acc.at[pl.ds(s * RPT + q * EB, EB)],
                              sz).wait()
    pltpu.make_async_copy(src_hbm.at[wid, pl.ds(0, CH)], src_v, sg0).wait()
    pltpu.make_async_copy(dst_hbm.at[wid, pl.ds(0, CH)], dst_v, sg1).wait()
    plsc.subcore_barrier()

    for q in range(NB // CH):
        if q > 0:
            pltpu.sync_copy(src_hbm.at[wid, pl.ds(q * CH, CH)], src_v)
            pltpu.sync_copy(dst_hbm.at[wid, pl.ds(q * CH, CH)], dst_v)
        pltpu.async_copy(hp_hbm.at[src_v.at[0]], rows0, sg0)

        def body(j, _):
            even = j % 2 == 0

            @pl.when(jnp.logical_and(even, j + 1 < CH))
            def _():
                pltpu.async_copy(hp_hbm.at[src_v.at[j + 1]], rows1, sg1)

            @pl.when(jnp.logical_and(jnp.logical_not(even), j + 1 < CH))
            def _():
                pltpu.async_copy(hp_hbm.at[src_v.at[j + 1]], rows0, sg0)

            @pl.when(even)
            def _():
                pltpu.make_async_copy(
                    hp_hbm.at[src_v.at[j]], rows0, sg0).wait()
                pltpu.sync_copy(rows0, acc.at[dst_v.at[j]], add=True)

            @pl.when(jnp.logical_not(even))
            def _():
                pltpu.make_async_copy(
                    hp_hbm.at[src_v.at[j]], rows1, sg1).wait()
                pltpu.sync_copy(rows1, acc.at[dst_v.at[j]], add=True)

            return 0

        lax.fori_loop(0, CH, body, 0, unroll=False)

    plsc.subcore_barrier()
    pltpu.sync_copy(acc.at[pl.ds(s * RPT, RPT)],
                    out_hbm.at[c, pl.ds(s * RPT, RPT)])


_RB = 640


def _prep_body(x_ref, w_ref, degp_ref, hp_ref, dis_ref):
    deg = jnp.sum(degp_ref[...], axis=0) + 1.0
    dis = lax.rsqrt(deg)[:, None]
    h = jnp.dot(x_ref[...], w_ref[...], preferred_element_type=jnp.float32)
    hp_ref[...] = h * dis
    dis_ref[...] = dis


def _prep_tc(x, w, degp):
    grid = (NP // _RB,)
    return pl.pallas_call(
        _prep_body,
        grid=grid,
        in_specs=[
            pl.BlockSpec((_RB, DIM), lambda i: (i, 0)),
            pl.BlockSpec((DIM, DIM), lambda i: (0, 0)),
            pl.BlockSpec((NW, _RB), lambda i: (0, i)),
        ],
        out_specs=[
            pl.BlockSpec((_RB, DIM), lambda i: (i, 0)),
            pl.BlockSpec((_RB, 1), lambda i: (i, 0)),
        ],
        out_shape=[
            jax.ShapeDtypeStruct((NP, DIM), jnp.float32),
            jax.ShapeDtypeStruct((NP, 1), jnp.float32),
        ],
    )(x, w, degp)


def _mid_body(p_ref, hp_ref, dis_ref, b_ref, w_ref, out_ref):
    z = (p_ref[0] + p_ref[1] + hp_ref[...]) * dis_ref[...] + b_ref[...]
    x2 = jnp.maximum(z, 0.0)
    h2 = jnp.dot(x2, w_ref[...], preferred_element_type=jnp.float32)
    out_ref[...] = h2 * dis_ref[...]


def _mid_tc(p, hp, dis, b, w):
    grid = (NP // _RB,)
    return pl.pallas_call(
        _mid_body,
        grid=grid,
        in_specs=[
            pl.BlockSpec((NC, _RB, DIM), lambda i: (0, i, 0)),
            pl.BlockSpec((_RB, DIM), lambda i: (i, 0)),
            pl.BlockSpec((_RB, 1), lambda i: (i, 0)),
            pl.BlockSpec((1, DIM), lambda i: (0, 0)),
            pl.BlockSpec((DIM, DIM), lambda i: (0, 0)),
        ],
        out_specs=pl.BlockSpec((_RB, DIM), lambda i: (i, 0)),
        out_shape=jax.ShapeDtypeStruct((NP, DIM), jnp.float32),
    )(p, hp, dis, b, w)


def _fin_body(p_ref, hp_ref, dis_ref, b_ref, out_ref):
    out_ref[...] = (p_ref[0] + p_ref[1] + hp_ref[...]) * dis_ref[...] \
        + b_ref[...]


_FB = 1000


def _fin_tc(p, hp, dis, b):
    grid = (N_NODES // _FB,)
    return pl.pallas_call(
        _fin_body,
        grid=grid,
        in_specs=[
            pl.BlockSpec((NC, _FB, DIM), lambda i: (0, i, 0)),
            pl.BlockSpec((_FB, DIM), lambda i: (i, 0)),
            pl.BlockSpec((_FB, 1), lambda i: (i, 0)),
            pl.BlockSpec((1, DIM), lambda i: (0, 0)),
        ],
        out_specs=pl.BlockSpec((_FB, DIM), lambda i: (i, 0)),
        out_shape=jax.ShapeDtypeStruct((N_NODES, DIM), jnp.float32),
    )(p, hp, dis, b)


@jax.jit
def kernel(x, edge_index, W1, b1, W2, b2):
    pad = N_NODES + jnp.arange(EP - N_EDGES, dtype=jnp.int32) % (NP - N_NODES)
    src = jnp.concatenate([edge_index[0].astype(jnp.int32), pad])
    dst = jnp.concatenate([edge_index[1].astype(jnp.int32), pad])
    src3d = src.reshape(NW, NB, EB)
    dst3d = dst.reshape(NW, NB, EB)
    xp = jnp.pad(x, ((0, NP - N_NODES), (0, 0)))

    degp = _deg_kernel(dst3d).reshape(NW, NP)
    hp1, dis = _prep_tc(xp, W1, degp)
    p1 = _agg_kernel(hp1, src3d, dst3d)
    hp2 = _mid_tc(p1, hp1, dis, b1.reshape(1, DIM), W2)
    p2 = _agg_kernel(hp2, src3d, dst3d)
    return _fin_tc(p2, hp2, dis, b2.reshape(1, DIM))
